# Initial kernel scaffold; baseline (speedup 1.0000x reference)
import jax
import jax.numpy as jnp
from jax.experimental import pallas as pl


def kernel(input_ids, attentions, hidden_states, output_vocabulary_probabilities, W_pg, b_pg, W_iv, b_iv):
    raise NotImplementedError("write your pallas kernel here")


# speedup vs baseline: 1.9202x; 1.9202x over previous
import functools

import jax
import jax.numpy as jnp
from jax import lax
from jax.experimental import pallas as pl
from jax.experimental.pallas import tpu as pltpu
from jax.experimental.pallas import tpu_sc as plsc

B, DEC, ENC, V = 4, 64, 512, 32000
H, A = 1024, 192
R = B * DEC
L = 16
G = ENC // L
NW = 32
ROWS_PER_W = R // NW
NBUF = 3


def _sigmoid(x):
    return 1.0 / (1.0 + jnp.exp(-x))


def _dense_body(att_ref, hid_ref, wiv_ref, biv_ref, wpg_ref, bpg_ref,
                add_ref, pg_ref):
    att = att_ref[...]
    iva = lax.dot_general(
        att, wiv_ref[...],
        dimension_numbers=(((2,), (0,)), ((), ())),
        preferred_element_type=jnp.float32)
    iva = _sigmoid(iva[..., 0] + biv_ref[0, 0])
    pg = _sigmoid(hid_ref[...] @ wpg_ref[...] + bpg_ref[0, 0])
    add_ref[...] = (1.0 - pg) * iva
    pg_ref[...] = jnp.broadcast_to(pg, (pg.shape[0], L))


def _dense(att3, hid2, W_iv, b_iv2, W_pg, b_pg2):
    RB = 8
    return pl.pallas_call(
        _dense_body,
        grid=(R // RB,),
        in_specs=[
            pl.BlockSpec((RB, ENC, A), lambda i: (i, 0, 0)),
            pl.BlockSpec((RB, H), lambda i: (i, 0)),
            pl.BlockSpec((A, 1), lambda i: (0, 0)),
            pl.BlockSpec((1, 1), lambda i: (0, 0)),
            pl.BlockSpec((H, 1), lambda i: (0, 0)),
            pl.BlockSpec((1, 1), lambda i: (0, 0)),
        ],
        out_specs=[
            pl.BlockSpec((RB, ENC), lambda i: (i, 0)),
            pl.BlockSpec((RB, L), lambda i: (i, 0)),
        ],
        out_shape=[
            jax.ShapeDtypeStruct((R, ENC), jnp.float32),
            jax.ShapeDtypeStruct((R, L), jnp.float32),
        ],
    )(att3, hid2, W_iv, b_iv2, W_pg, b_pg2)


def _jumps_body(ids_ref, jumps_ref, valids_ref, first_ref):
    ids = ids_ref[...]
    n = ids.shape[0]
    eq = ids[:, :, None] == ids[:, None, :]
    jj = lax.broadcasted_iota(jnp.int32, (n, L, L), 2)
    ii = lax.broadcasted_iota(jnp.int32, (n, L, L), 1)
    rank = jnp.sum(jnp.where(eq & (jj < ii), 1, 0), axis=2)
    first_ref[...] = (rank == 0).astype(jnp.int32)
    lane = lax.broadcasted_iota(jnp.int32, (n, L), 1)
    for k in range(4):
        d = 1 << k
        cond = eq & (rank[:, None, :] == rank[:, :, None] + d)
        valid = jnp.sum(jnp.where(cond, 1, 0), axis=2) > 0
        jump = jnp.sum(jnp.where(cond, jj, 0), axis=2)
        jumps_ref[k] = jnp.where(valid, jump, lane)
        valids_ref[k] = valid.astype(jnp.int32)


def _jumps(ids16):
    return pl.pallas_call(
        _jumps_body,
        out_shape=[
            jax.ShapeDtypeStruct((4, B * G, L), jnp.int32),
            jax.ShapeDtypeStruct((4, B * G, L), jnp.int32),
            jax.ShapeDtypeStruct((B * G, L), jnp.int32),
        ],
    )(ids16)


def _make_sc_kernel():
    mesh = plsc.VectorSubcoreMesh(core_axis_name="c", subcore_axis_name="s")
    scratch = (
        [pltpu.VMEM((V,), jnp.float32) for _ in range(NBUF)]
        + [
            pltpu.VMEM((ROWS_PER_W, ENC), jnp.float32),
            pltpu.VMEM((ROWS_PER_W, L), jnp.float32),
            pltpu.VMEM((ENC,), jnp.int32),
            pltpu.VMEM((4 * ENC,), jnp.int32),
            pltpu.VMEM((4 * ENC,), jnp.int32),
            pltpu.VMEM((ENC,), jnp.int32),
            pltpu.VMEM((L,), jnp.float32),
        ]
        + [pltpu.SemaphoreType.DMA] * (2 * NBUF)
    )

    @functools.partial(
        pl.kernel,
        mesh=mesh,
        out_type=jax.ShapeDtypeStruct((R, V), jnp.float32),
        scratch_types=scratch,
        compiler_params=pltpu.CompilerParams(needs_layout_passes=False),
    )
    def sc_kernel(ovp_hbm, pg_hbm, add_hbm, idx_hbm, jumps_hbm, valids_hbm,
                  first_hbm, out_hbm, *scr):
        bufs = scr[:NBUF]
        add_v, pg_v, idx_v, jumps_v, valids_v, first_v, sc16 = scr[NBUF:NBUF + 7]
        in_sems = scr[NBUF + 7:NBUF + 7 + NBUF]
        out_sems = scr[NBUF + 7 + NBUF:]

        wid = lax.axis_index("s") * 2 + lax.axis_index("c")
        base = wid * ROWS_PER_W
        b = wid // (NW // B)

        pltpu.sync_copy(add_hbm.at[pl.ds(base, ROWS_PER_W)], add_v)
        pltpu.sync_copy(pg_hbm.at[pl.ds(base, ROWS_PER_W)], pg_v)
        pltpu.sync_copy(idx_hbm.at[b], idx_v)
        pltpu.sync_copy(jumps_hbm.at[b], jumps_v)
        pltpu.sync_copy(valids_hbm.at[b], valids_v)
        pltpu.sync_copy(first_hbm.at[b], first_v)

        in_descs = [None] * ROWS_PER_W
        out_descs = [None] * ROWS_PER_W

        def start_in(r):
            in_descs[r] = pltpu.async_copy(
                ovp_hbm.at[base + r], bufs[r % NBUF], in_sems[r % NBUF])

        for r in range(NBUF):
            start_in(r)

        for i in range(ROWS_PER_W):
            row = bufs[i % NBUF]
            in_descs[i].wait()

            pgv = pg_v[i, :]

            @plsc.parallel_loop(0, V, L, unroll=16)
            def _scale(off):
                row[pl.ds(off, L)] = row[pl.ds(off, L)] * pgv

            def scatter_group(g, carry):
                e0 = g * L
                il = idx_v[pl.ds(e0, L)]
                v = add_v[i, pl.ds(e0, L)]
                plsc.addupdate_scatter(row, [il], v)
                return carry

            lax.fori_loop(0, G, scatter_group, 0)

            out_descs[i] = pltpu.async_copy(
                row, out_hbm.at[base + i], out_sems[i % NBUF])
            if i + NBUF < ROWS_PER_W:
                out_descs[i].wait()
                start_in(i + NBUF)

        for i in range(ROWS_PER_W - NBUF, ROWS_PER_W):
            if i >= 0:
                out_descs[i].wait()

    return sc_kernel


_sc_kernel = _make_sc_kernel()


def kernel(input_ids, attentions, hidden_states, output_vocabulary_probabilities,
           W_pg, b_pg, W_iv, b_iv):
    att3 = attentions.reshape(R, ENC, A)
    hid2 = hidden_states.reshape(R, H)
    ovp2 = output_vocabulary_probabilities.reshape(R, V)
    ids16 = input_ids.reshape(B * G, L)
    b_iv2 = b_iv.reshape(1, 1)
    b_pg2 = b_pg.reshape(1, 1)

    add2, pg16 = _dense(att3, hid2, W_iv, b_iv2, W_pg, b_pg2)
    jumps, valids, first = _jumps(ids16)

    jumps_b = jumps.reshape(4, B, G, L).transpose(1, 0, 2, 3).reshape(B, 4 * ENC)
    valids_b = valids.reshape(4, B, G, L).transpose(1, 0, 2, 3).reshape(B, 4 * ENC)
    first_b = first.reshape(B, ENC)

    out2 = _sc_kernel(ovp2, pg16, add2, input_ids, jumps_b, valids_b, first_b)
    return out2.reshape(B, DEC, V)


# speedup vs baseline: 2.0192x; 1.0516x over previous
import functools

import jax
import jax.numpy as jnp
from jax import lax
from jax.experimental import pallas as pl
from jax.experimental.pallas import tpu as pltpu
from jax.experimental.pallas import tpu_sc as plsc

B, DEC, ENC, V = 4, 64, 512, 32000
H, A = 1024, 192
R = B * DEC
L = 16
G = ENC // L
NW = 32
ROWS_PER_W = R // NW
NBUF = 3


def _sigmoid(x):
    return 1.0 / (1.0 + jnp.exp(-x))


def _dense_body(att_ref, hid_ref, wiv_ref, biv_ref, wpg_ref, bpg_ref,
                add_ref, pg_ref):
    att = att_ref[...]
    iva = lax.dot_general(
        att, wiv_ref[...],
        dimension_numbers=(((2,), (0,)), ((), ())),
        preferred_element_type=jnp.float32)
    iva = _sigmoid(iva[..., 0] + biv_ref[0, 0])
    pg = _sigmoid(hid_ref[...] @ wpg_ref[...] + bpg_ref[0, 0])
    add_ref[...] = (1.0 - pg) * iva
    pg_ref[...] = jnp.broadcast_to(pg, (pg.shape[0], L))


def _dense(att3, hid2, W_iv, b_iv2, W_pg, b_pg2):
    RB = 8
    return pl.pallas_call(
        _dense_body,
        grid=(R // RB,),
        in_specs=[
            pl.BlockSpec((RB, ENC, A), lambda i: (i, 0, 0)),
            pl.BlockSpec((RB, H), lambda i: (i, 0)),
            pl.BlockSpec((A, 1), lambda i: (0, 0)),
            pl.BlockSpec((1, 1), lambda i: (0, 0)),
            pl.BlockSpec((H, 1), lambda i: (0, 0)),
            pl.BlockSpec((1, 1), lambda i: (0, 0)),
        ],
        out_specs=[
            pl.BlockSpec((RB, ENC), lambda i: (i, 0)),
            pl.BlockSpec((RB, L), lambda i: (i, 0)),
        ],
        out_shape=[
            jax.ShapeDtypeStruct((R, ENC), jnp.float32),
            jax.ShapeDtypeStruct((R, L), jnp.float32),
        ],
    )(att3, hid2, W_iv, b_iv2, W_pg, b_pg2)


def _make_sc_kernel():
    mesh = plsc.VectorSubcoreMesh(core_axis_name="c", subcore_axis_name="s")
    scratch = (
        [pltpu.VMEM((V,), jnp.float32) for _ in range(NBUF)]
        + [
            pltpu.VMEM((ROWS_PER_W, ENC), jnp.float32),
            pltpu.VMEM((ROWS_PER_W, L), jnp.float32),
            pltpu.VMEM((ENC,), jnp.int32),
        ]
        + [pltpu.SemaphoreType.DMA] * (2 * NBUF)
    )

    @functools.partial(
        pl.kernel,
        mesh=mesh,
        out_type=jax.ShapeDtypeStruct((R, V), jnp.float32),
        scratch_types=scratch,
        compiler_params=pltpu.CompilerParams(needs_layout_passes=False),
    )
    def sc_kernel(ovp_hbm, pg_hbm, add_hbm, idx_hbm, out_hbm, *scr):
        bufs = scr[:NBUF]
        add_v, pg_v, idx_v = scr[NBUF:NBUF + 3]
        in_sems = scr[NBUF + 3:NBUF + 3 + NBUF]
        out_sems = scr[NBUF + 3 + NBUF:]

        wid = lax.axis_index("s") * 2 + lax.axis_index("c")
        base = wid * ROWS_PER_W
        b = wid // (NW // B)

        pltpu.sync_copy(add_hbm.at[pl.ds(base, ROWS_PER_W)], add_v)
        pltpu.sync_copy(pg_hbm.at[pl.ds(base, ROWS_PER_W)], pg_v)
        pltpu.sync_copy(idx_hbm.at[b], idx_v)

        in_descs = [None] * ROWS_PER_W
        out_descs = [None] * ROWS_PER_W

        def start_in(r):
            in_descs[r] = pltpu.async_copy(
                ovp_hbm.at[base + r], bufs[r % NBUF], in_sems[r % NBUF])

        for r in range(NBUF):
            start_in(r)

        for i in range(ROWS_PER_W):
            row = bufs[i % NBUF]
            in_descs[i].wait()

            pgv = pg_v[i, :]

            @plsc.parallel_loop(0, V, L, unroll=16)
            def _scale(off):
                row[pl.ds(off, L)] = row[pl.ds(off, L)] * pgv

            def scatter_group(g, carry):
                e0 = g * L
                il = idx_v[pl.ds(e0, L)]
                v = add_v[i, pl.ds(e0, L)]
                plsc.addupdate_scatter(row, [il], v)
                return carry

            lax.fori_loop(0, G, scatter_group, 0)

            out_descs[i] = pltpu.async_copy(
                row, out_hbm.at[base + i], out_sems[i % NBUF])
            if i + NBUF < ROWS_PER_W:
                out_descs[i].wait()
                start_in(i + NBUF)

        for i in range(ROWS_PER_W - NBUF, ROWS_PER_W):
            if i >= 0:
                out_descs[i].wait()

    return sc_kernel


_sc_kernel = _make_sc_kernel()


def kernel(input_ids, attentions, hidden_states, output_vocabulary_probabilities,
           W_pg, b_pg, W_iv, b_iv):
    att3 = attentions.reshape(R, ENC, A)
    hid2 = hidden_states.reshape(R, H)
    ovp2 = output_vocabulary_probabilities.reshape(R, V)
    b_iv2 = b_iv.reshape(1, 1)
    b_pg2 = b_pg.reshape(1, 1)

    add2, pg16 = _dense(att3, hid2, W_iv, b_iv2, W_pg, b_pg2)
    out2 = _sc_kernel(ovp2, pg16, add2, input_ids)
    return out2.reshape(B, DEC, V)


# speedup vs baseline: 4.1312x; 2.0459x over previous
import functools

import jax
import jax.numpy as jnp
from jax import lax
from jax.experimental import pallas as pl
from jax.experimental.pallas import tpu as pltpu
from jax.experimental.pallas import tpu_sc as plsc

B, DEC, ENC, V = 4, 64, 512, 32000
H, A = 1024, 192
R = B * DEC
L = 16
G = ENC // L
NW = 32
SPLIT = 160
NBUF = 3


def _sigmoid(x):
    return 1.0 / (1.0 + jnp.exp(-x))


def _dense_body(att_ref, hid_ref, wiv_ref, biv_ref, wpg_ref, bpg_ref,
                add_ref, pg_ref):
    att = att_ref[...]
    iva = jnp.sum(att * wiv_ref[...][None, :, :], axis=1)
    iva = _sigmoid(iva + biv_ref[0, 0])
    pg = _sigmoid(hid_ref[...] @ wpg_ref[...] + bpg_ref[0, 0])
    add_ref[...] = (1.0 - pg) * iva
    pg_ref[...] = jnp.broadcast_to(pg, (pg.shape[0], 128))


def _dense_chunk(row0, nrows, att3, hid2, W_iv, b_iv2, W_pg, b_pg2):
    RB = 8
    off = row0 // RB
    return pl.pallas_call(
        _dense_body,
        grid=(nrows // RB,),
        in_specs=[
            pl.BlockSpec((RB, A, ENC), lambda i: (i + off, 0, 0)),
            pl.BlockSpec((RB, H), lambda i: (i + off, 0)),
            pl.BlockSpec((A, 1), lambda i: (0, 0)),
            pl.BlockSpec((1, 1), lambda i: (0, 0)),
            pl.BlockSpec((H, 1), lambda i: (0, 0)),
            pl.BlockSpec((1, 1), lambda i: (0, 0)),
        ],
        out_specs=[
            pl.BlockSpec((RB, ENC), lambda i: (i, 0)),
            pl.BlockSpec((RB, 128), lambda i: (i, 0)),
        ],
        out_shape=[
            jax.ShapeDtypeStruct((nrows, ENC), jnp.float32),
            jax.ShapeDtypeStruct((nrows, 128), jnp.float32),
        ],
        name=f"dense_rows{row0}",
    )(att3, hid2, W_iv, b_iv2, W_pg, b_pg2)


def _make_sc_kernel(row0, nrows, makes_output):
    rows_w = nrows // NW
    mesh = plsc.VectorSubcoreMesh(core_axis_name="c", subcore_axis_name="s")
    scratch = (
        [pltpu.VMEM((V,), jnp.float32) for _ in range(NBUF)]
        + [
            pltpu.VMEM((rows_w * ENC,), jnp.float32),
            pltpu.VMEM((rows_w * 128,), jnp.float32),
            pltpu.VMEM((B * ENC,), jnp.int32),
        ]
        + [pltpu.SemaphoreType.DMA] * (2 * NBUF + 1)
    )

    @functools.partial(
        pl.kernel,
        mesh=mesh,
        out_type=(jax.ShapeDtypeStruct((R, V), jnp.float32)
                  if makes_output else ()),
        scratch_types=scratch,
        compiler_params=pltpu.CompilerParams(needs_layout_passes=False),
        name=f"sc_scatter_rows{row0}",
    )
    def sc_kernel(ovp_hbm, pg_hbm, add_hbm, idx_hbm, out_hbm, *scr):
        bufs = scr[:NBUF]
        add_v, pg_v, idx_v = scr[NBUF:NBUF + 3]
        in_sems = scr[NBUF + 3:NBUF + 3 + NBUF]
        out_sems = scr[NBUF + 3 + NBUF:NBUF + 3 + 2 * NBUF]
        pre_sem = scr[NBUF + 3 + 2 * NBUF]

        wid = lax.axis_index("s") * 2 + lax.axis_index("c")
        base = wid * rows_w

        pres = []
        for i in range(rows_w):
            pres.append(pltpu.async_copy(
                add_hbm.at[base + i], add_v.at[pl.ds(i * ENC, ENC)], pre_sem))
            pres.append(pltpu.async_copy(
                pg_hbm.at[base + i], pg_v.at[pl.ds(i * 128, 128)], pre_sem))
        pres.append(pltpu.async_copy(idx_hbm, idx_v, pre_sem))

        in_descs = [None] * rows_w
        out_descs = [None] * rows_w

        def start_in(r):
            in_descs[r] = pltpu.async_copy(
                ovp_hbm.at[row0 + base + r], bufs[r % NBUF], in_sems[r % NBUF])

        for r in range(min(NBUF, rows_w)):
            start_in(r)

        for p in pres:
            p.wait()

        for i in range(rows_w):
            row = bufs[i % NBUF]
            in_descs[i].wait()

            pgv = pg_v[pl.ds(i * 128, L)]

            @plsc.parallel_loop(0, V, L, unroll=16)
            def _scale(off):
                row[pl.ds(off, L)] = row[pl.ds(off, L)] * pgv

            b_i = (row0 + base + i) // DEC

            def scatter_group(g, carry):
                e0 = g * L
                il = idx_v[pl.ds(b_i * ENC + e0, L)]
                v = add_v[pl.ds(i * ENC + e0, L)]
                plsc.addupdate_scatter(row, [il], v)
                return carry

            lax.fori_loop(0, G, scatter_group, 0)

            out_descs[i] = pltpu.async_copy(
                row, out_hbm.at[row0 + base + i], out_sems[i % NBUF])
            if i + NBUF < rows_w:
                out_descs[i].wait()
                start_in(i + NBUF)

        for i in range(max(0, rows_w - NBUF), rows_w):
            out_descs[i].wait()

    return sc_kernel


_sc_kernel0 = _make_sc_kernel(0, SPLIT, True)
_sc_kernel1 = _make_sc_kernel(SPLIT, R - SPLIT, False)


def kernel(input_ids, attentions, hidden_states, output_vocabulary_probabilities,
           W_pg, b_pg, W_iv, b_iv):
    att3 = attentions.transpose(0, 1, 3, 2).reshape(R, A, ENC)
    hid2 = hidden_states.reshape(R, H)
    ovp2 = output_vocabulary_probabilities.reshape(R, V)
    b_iv2 = b_iv.reshape(1, 1)
    b_pg2 = b_pg.reshape(1, 1)

    add0, pg0 = _dense_chunk(0, SPLIT, att3, hid2, W_iv, b_iv2, W_pg, b_pg2)
    ids_flat = input_ids.reshape(B * ENC)
    out_full = _sc_kernel0(ovp2, pg0, add0, ids_flat)
    out_ref = jax.new_ref(out_full)
    add1, pg1 = _dense_chunk(SPLIT, R - SPLIT, att3, hid2, W_iv, b_iv2,
                             W_pg, b_pg2)
    _sc_kernel1(ovp2, pg1, add1, ids_flat, out_ref)
    return jax.freeze(out_ref).reshape(B, DEC, V)


# speedup vs baseline: 4.1505x; 1.0047x over previous
import functools

import jax
import jax.numpy as jnp
from jax import lax
from jax.experimental import pallas as pl
from jax.experimental.pallas import tpu as pltpu
from jax.experimental.pallas import tpu_sc as plsc

B, DEC, ENC, V = 4, 64, 512, 32000
H, A = 1024, 192
R = B * DEC
L = 16
G = ENC // L
NW = 32
SPLIT = 128
NBUF = 3


def _sigmoid(x):
    return 1.0 / (1.0 + jnp.exp(-x))


def _dense_body(att_ref, hid_ref, wiv_ref, biv_ref, wpg_ref, bpg_ref,
                add_ref, pg_ref):
    att = att_ref[...]
    iva = jnp.sum(att * wiv_ref[...][None, :, :], axis=1)
    iva = _sigmoid(iva + biv_ref[0, 0])
    pg = _sigmoid(hid_ref[...] @ wpg_ref[...] + bpg_ref[0, 0])
    add_ref[...] = (1.0 - pg) * iva
    pg_ref[...] = jnp.broadcast_to(pg, (pg.shape[0], 128))


def _dense_chunk(row0, nrows, att3, hid2, W_iv, b_iv2, W_pg, b_pg2):
    RB = 8
    off = row0 // RB
    return pl.pallas_call(
        _dense_body,
        grid=(nrows // RB,),
        in_specs=[
            pl.BlockSpec((RB, A, ENC), lambda i: (i + off, 0, 0)),
            pl.BlockSpec((RB, H), lambda i: (i + off, 0)),
            pl.BlockSpec((A, 1), lambda i: (0, 0)),
            pl.BlockSpec((1, 1), lambda i: (0, 0)),
            pl.BlockSpec((H, 1), lambda i: (0, 0)),
            pl.BlockSpec((1, 1), lambda i: (0, 0)),
        ],
        out_specs=[
            pl.BlockSpec((RB, ENC), lambda i: (i, 0)),
            pl.BlockSpec((RB, 128), lambda i: (i, 0)),
        ],
        out_shape=[
            jax.ShapeDtypeStruct((nrows, ENC), jnp.float32),
            jax.ShapeDtypeStruct((nrows, 128), jnp.float32),
        ],
        name=f"dense_rows{row0}",
    )(att3, hid2, W_iv, b_iv2, W_pg, b_pg2)


def _make_sc_kernel(row0, nrows, makes_output):
    rows_w = nrows // NW
    mesh = plsc.VectorSubcoreMesh(core_axis_name="c", subcore_axis_name="s")
    scratch = (
        [pltpu.VMEM((V,), jnp.float32) for _ in range(NBUF)]
        + [
            pltpu.VMEM((rows_w * ENC,), jnp.float32),
            pltpu.VMEM((rows_w * 128,), jnp.float32),
            pltpu.VMEM((B * ENC,), jnp.int32),
        ]
        + [pltpu.SemaphoreType.DMA] * (2 * NBUF + 1)
    )

    @functools.partial(
        pl.kernel,
        mesh=mesh,
        out_type=(jax.ShapeDtypeStruct((R, V), jnp.float32)
                  if makes_output else ()),
        scratch_types=scratch,
        compiler_params=pltpu.CompilerParams(needs_layout_passes=False),
        name=f"sc_scatter_rows{row0}",
    )
    def sc_kernel(ovp_hbm, pg_hbm, add_hbm, idx_hbm, out_hbm, *scr):
        bufs = scr[:NBUF]
        add_v, pg_v, idx_v = scr[NBUF:NBUF + 3]
        in_sems = scr[NBUF + 3:NBUF + 3 + NBUF]
        out_sems = scr[NBUF + 3 + NBUF:NBUF + 3 + 2 * NBUF]
        pre_sem = scr[NBUF + 3 + 2 * NBUF]

        wid = lax.axis_index("s") * 2 + lax.axis_index("c")
        base = wid * rows_w

        pres = []
        for i in range(rows_w):
            pres.append(pltpu.async_copy(
                add_hbm.at[base + i], add_v.at[pl.ds(i * ENC, ENC)], pre_sem))
            pres.append(pltpu.async_copy(
                pg_hbm.at[base + i], pg_v.at[pl.ds(i * 128, 128)], pre_sem))
        pres.append(pltpu.async_copy(idx_hbm, idx_v, pre_sem))

        in_descs = [None] * rows_w
        out_descs = [None] * rows_w

        def start_in(r):
            in_descs[r] = pltpu.async_copy(
                ovp_hbm.at[row0 + base + r], bufs[r % NBUF], in_sems[r % NBUF])

        for r in range(min(NBUF, rows_w)):
            start_in(r)

        for p in pres:
            p.wait()

        for i in range(rows_w):
            row = bufs[i % NBUF]
            in_descs[i].wait()

            pgv = pg_v[pl.ds(i * 128, L)]

            @plsc.parallel_loop(0, V, L, unroll=16)
            def _scale(off):
                row[pl.ds(off, L)] = row[pl.ds(off, L)] * pgv

            b_i = (row0 + base + i) // DEC

            def scatter_group(g, carry):
                e0 = g * L
                il = idx_v[pl.ds(b_i * ENC + e0, L)]
                v = add_v[pl.ds(i * ENC + e0, L)]
                plsc.addupdate_scatter(row, [il], v)
                return carry

            lax.fori_loop(0, G, scatter_group, 0)

            out_descs[i] = pltpu.async_copy(
                row, out_hbm.at[row0 + base + i], out_sems[i % NBUF])
            if i + NBUF < rows_w:
                out_descs[i].wait()
                start_in(i + NBUF)

        for i in range(max(0, rows_w - NBUF), rows_w):
            out_descs[i].wait()

    return sc_kernel


_sc_kernel0 = _make_sc_kernel(0, SPLIT, True)
_sc_kernel1 = _make_sc_kernel(SPLIT, R - SPLIT, False)


def kernel(input_ids, attentions, hidden_states, output_vocabulary_probabilities,
           W_pg, b_pg, W_iv, b_iv):
    att3 = attentions.transpose(0, 1, 3, 2).reshape(R, A, ENC)
    hid2 = hidden_states.reshape(R, H)
    ovp2 = output_vocabulary_probabilities.reshape(R, V)
    b_iv2 = b_iv.reshape(1, 1)
    b_pg2 = b_pg.reshape(1, 1)

    add0, pg0 = _dense_chunk(0, SPLIT, att3, hid2, W_iv, b_iv2, W_pg, b_pg2)
    ids_flat = input_ids.reshape(B * ENC)
    out_full = _sc_kernel0(ovp2, pg0, add0, ids_flat)
    out_ref = jax.new_ref(out_full)
    add1, pg1 = _dense_chunk(SPLIT, R - SPLIT, att3, hid2, W_iv, b_iv2,
                             W_pg, b_pg2)
    _sc_kernel1(ovp2, pg1, add1, ids_flat, out_ref)
    return jax.freeze(out_ref).reshape(B, DEC, V)
